# baseline (speedup 1.0000x reference)
import functools

import jax
import jax.numpy as jnp
from jax.experimental import pallas as pl

_BLOCK = 2000


def _mlp_head_kernel(x_ref, w1_ref, b1_ref, wh_ref, bh_ref, wv_ref, bv_ref,
                     a_ref, v_ref):
    x = x_ref[...]
    y = jnp.tanh(
        jnp.dot(x, w1_ref[...], preferred_element_type=jnp.float32)
        + b1_ref[...])
    logits = (jnp.dot(y, wh_ref[...], preferred_element_type=jnp.float32)
              + bh_ref[...])
    m = jnp.max(logits, axis=-1, keepdims=True)
    s = logits - m
    lse = jnp.log(jnp.sum(jnp.exp(s), axis=-1, keepdims=True))
    a_ref[...] = s - lse
    v_ref[...] = (jnp.dot(y, wv_ref[...], preferred_element_type=jnp.float32)
                  + bv_ref[...])


@functools.partial(jax.jit, static_argnames=())
def kernel(after_comm, W1, b1, Wh, bh, Wv, bv):
    n = after_comm.shape[0]
    x = after_comm.reshape(n, -1)
    d_in = x.shape[1]
    hid = W1.shape[1]
    n_act = Wh.shape[1]
    b = _BLOCK
    grid = (n // b,)

    a, v = pl.pallas_call(
        _mlp_head_kernel,
        grid=grid,
        in_specs=[
            pl.BlockSpec((b, d_in), lambda i: (i, 0)),
            pl.BlockSpec((d_in, hid), lambda i: (0, 0)),
            pl.BlockSpec((1, hid), lambda i: (0, 0)),
            pl.BlockSpec((hid, n_act), lambda i: (0, 0)),
            pl.BlockSpec((1, n_act), lambda i: (0, 0)),
            pl.BlockSpec((hid, 1), lambda i: (0, 0)),
            pl.BlockSpec((1, 1), lambda i: (0, 0)),
        ],
        out_specs=[
            pl.BlockSpec((b, n_act), lambda i: (i, 0)),
            pl.BlockSpec((b, 1), lambda i: (i, 0)),
        ],
        out_shape=[
            jax.ShapeDtypeStruct((n, n_act), jnp.float32),
            jax.ShapeDtypeStruct((n, 1), jnp.float32),
        ],
    )(x, W1, b1.reshape(1, hid), Wh, bh.reshape(1, n_act),
      Wv, bv.reshape(1, 1))
    return (a, v)
